# baseline (speedup 1.0000x reference)
import jax
import jax.numpy as jnp
from jax.experimental import pallas as pl
from jax.experimental.pallas import tpu as pltpu

CHUNKS = (512, 1024, 2048, 4096, 8192, 8192, 8192, 512)
MAX_CHUNK = max(CHUNKS)
N_BUF = 2


def _gating_body(x_hbm, w1_ref, b1_ref, w2_ref, b2_ref, out_ref, out2_ref,
                 buf0, buf1, sem0, sem1):
    bufs = (buf0, buf1)
    sems = (sem0, sem1)
    w1b = w1_ref[...].astype(jnp.bfloat16)
    w2b = w2_ref[...].astype(jnp.bfloat16)
    b1 = b1_ref[...]
    b2 = b2_ref[...]

    offs = []
    o = 0
    for c in CHUNKS:
        offs.append(o)
        o += c

    def copy_in(i):
        c = CHUNKS[i]
        return pltpu.make_async_copy(
            x_hbm.at[pl.ds(offs[i], c), :],
            bufs[i % N_BUF].at[pl.ds(0, c), :],
            sems[i % N_BUF],
        )

    copy_in(0).start()
    for i, c in enumerate(CHUNKS):
        if i + 1 < len(CHUNKS):
            copy_in(i + 1).start()
        copy_in(i).wait()
        xb = bufs[i % N_BUF][pl.ds(0, c), :].astype(jnp.bfloat16)
        h_t = jnp.tanh(
            jax.lax.dot_general(w1b, xb, (((1,), (1,)), ((), ())),
                                preferred_element_type=jnp.float32)
            + b1
        )
        gates_t = (
            jax.lax.dot_general(w2b, h_t.astype(jnp.bfloat16),
                                (((1,), (0,)), ((), ())),
                                preferred_element_type=jnp.float32)
            + b2
        )
        out_ref[:, pl.ds(offs[i], c)] = gates_t
        out2_ref[:, pl.ds(offs[i], c)] = gates_t


@jax.jit
def _gating(x, w1, b1, w2, b2):
    tokens, feats = x.shape
    num_experts = w1.shape[0]
    gates_t = pl.pallas_call(
        _gating_body,
        in_specs=[
            pl.BlockSpec(memory_space=pltpu.MemorySpace.HBM),
            pl.BlockSpec((num_experts, feats), lambda: (0, 0)),
            pl.BlockSpec((num_experts, 1), lambda: (0, 0)),
            pl.BlockSpec((num_experts, num_experts), lambda: (0, 0)),
            pl.BlockSpec((num_experts, 1), lambda: (0, 0)),
        ],
        out_specs=[
            pl.BlockSpec((num_experts, tokens), lambda: (0, 0)),
            pl.BlockSpec((num_experts, tokens), lambda: (0, 0)),
        ],
        out_shape=[
            jax.ShapeDtypeStruct((num_experts, tokens), jnp.float32),
            jax.ShapeDtypeStruct((num_experts, tokens), jnp.float32),
        ],
        scratch_shapes=[
            pltpu.VMEM((MAX_CHUNK, 768), jnp.float32),
            pltpu.VMEM((MAX_CHUNK, 768), jnp.float32),
            pltpu.SemaphoreType.DMA,
            pltpu.SemaphoreType.DMA,
        ],
    )(x, w1, b1, w2, b2)
    return gates_t[0].T, gates_t[1].T


def kernel(x, W1, b1, W2, b2, train):
    out, gates = _gating(x, W1, b1.reshape(-1, 1), W2, b2.reshape(-1, 1))
    return (out, gates)


# speedup vs baseline: 1.0380x; 1.0380x over previous
import jax
import jax.numpy as jnp
from jax.experimental import pallas as pl
from jax.experimental.pallas import tpu as pltpu

CHUNKS = (512, 1024, 2048, 4096, 4096, 4096, 4096, 4096, 4096, 4096, 512)
MAX_CHUNK = max(CHUNKS)
N_BUF = 3


def _gating_body(x_hbm, w1_ref, b1_ref, w2_ref, b2_ref, out_ref, out2_ref,
                 buf0, buf1, buf2, sem0, sem1, sem2):
    bufs = (buf0, buf1, buf2)
    sems = (sem0, sem1, sem2)
    w1b = w1_ref[...].astype(jnp.bfloat16)
    w2b = w2_ref[...].astype(jnp.bfloat16)
    b1 = b1_ref[...]
    b2 = b2_ref[...]

    offs = []
    o = 0
    for c in CHUNKS:
        offs.append(o)
        o += c

    def copy_in(i):
        c = CHUNKS[i]
        return pltpu.make_async_copy(
            x_hbm.at[pl.ds(offs[i], c), :],
            bufs[i % N_BUF].at[pl.ds(0, c), :],
            sems[i % N_BUF],
        )

    copy_in(0).start()
    copy_in(1).start()
    for i, c in enumerate(CHUNKS):
        if i + 2 < len(CHUNKS):
            copy_in(i + 2).start()
        copy_in(i).wait()
        xb = bufs[i % N_BUF][pl.ds(0, c), :].astype(jnp.bfloat16)
        h_t = jnp.tanh(
            jax.lax.dot_general(w1b, xb, (((1,), (1,)), ((), ())),
                                preferred_element_type=jnp.float32)
            + b1
        )
        gates_t = (
            jax.lax.dot_general(w2b, h_t.astype(jnp.bfloat16),
                                (((1,), (0,)), ((), ())),
                                preferred_element_type=jnp.float32)
            + b2
        )
        out_ref[:, pl.ds(offs[i], c)] = gates_t
        out2_ref[:, pl.ds(offs[i], c)] = gates_t


@jax.jit
def _gating(x, w1, b1, w2, b2):
    tokens, feats = x.shape
    num_experts = w1.shape[0]
    gates_t = pl.pallas_call(
        _gating_body,
        in_specs=[
            pl.BlockSpec(memory_space=pltpu.MemorySpace.HBM),
            pl.BlockSpec((num_experts, feats), lambda: (0, 0)),
            pl.BlockSpec((num_experts, 1), lambda: (0, 0)),
            pl.BlockSpec((num_experts, num_experts), lambda: (0, 0)),
            pl.BlockSpec((num_experts, 1), lambda: (0, 0)),
        ],
        out_specs=[
            pl.BlockSpec((num_experts, tokens), lambda: (0, 0)),
            pl.BlockSpec((num_experts, tokens), lambda: (0, 0)),
        ],
        out_shape=[
            jax.ShapeDtypeStruct((num_experts, tokens), jnp.float32),
            jax.ShapeDtypeStruct((num_experts, tokens), jnp.float32),
        ],
        scratch_shapes=[
            pltpu.VMEM((MAX_CHUNK, 768), jnp.float32),
            pltpu.VMEM((MAX_CHUNK, 768), jnp.float32),
            pltpu.VMEM((MAX_CHUNK, 768), jnp.float32),
            pltpu.SemaphoreType.DMA,
            pltpu.SemaphoreType.DMA,
            pltpu.SemaphoreType.DMA,
        ],
    )(x, w1, b1, w2, b2)
    return gates_t[0].T, gates_t[1].T


def kernel(x, W1, b1, W2, b2, train):
    out, gates = _gating(x, W1, b1.reshape(-1, 1), W2, b2.reshape(-1, 1))
    return (out, gates)
